# baseline (speedup 1.0000x reference)
import functools

import jax
import jax.numpy as jnp
from jax import lax
from jax.experimental import pallas as pl
from jax.experimental.pallas import tpu as pltpu
from jax.experimental.pallas import tpu_sc as plsc

N = 10000
E = 320000
NODE_F = 11
EDGE_F = 14
H = 128
P = 128
L = 3
G = 16

NC = 2
NS = 16
NW = NC * NS
EPW = E // NW
C = 80
NCHUNK = EPW // C
NP = 10240
RPT = NP // NS

_f32 = jnp.float32


def _embed_body(x_ref, w_ref, b_ref, o_ref):
    o_ref[...] = (
        jnp.dot(x_ref[...], w_ref[...], preferred_element_type=_f32) + b_ref[...]
    )


def _embed(x, w, b):
    bn = 2000
    return pl.pallas_call(
        _embed_body,
        grid=(N // bn,),
        in_specs=[
            pl.BlockSpec((bn, NODE_F), lambda i: (i, 0)),
            pl.BlockSpec((NODE_F, H), lambda i: (0, 0)),
            pl.BlockSpec((1, H), lambda i: (0, 0)),
        ],
        out_specs=pl.BlockSpec((bn, H), lambda i: (i, 0)),
        out_shape=jax.ShapeDtypeStruct((N, H), _f32),
    )(x, w, b.reshape(1, H))


def _ea_body(a_ref, we_ref, be_ref, wl_ref, bl_ref, o_ref):
    t = jnp.dot(a_ref[...], we_ref[...], preferred_element_type=_f32) + be_ref[...]
    o_ref[...] = jnp.dot(t, wl_ref[...], preferred_element_type=_f32) + bl_ref[...]


def _edge_linear(edge_attr, we, be, wl, bl):
    bn = 4000
    return pl.pallas_call(
        _ea_body,
        grid=(E // bn,),
        in_specs=[
            pl.BlockSpec((bn, EDGE_F), lambda i: (i, 0)),
            pl.BlockSpec((EDGE_F, H), lambda i: (0, 0)),
            pl.BlockSpec((1, H), lambda i: (0, 0)),
            pl.BlockSpec((H, H), lambda i: (0, 0)),
            pl.BlockSpec((1, H), lambda i: (0, 0)),
        ],
        out_specs=pl.BlockSpec((bn, H), lambda i: (i, 0)),
        out_shape=jax.ShapeDtypeStruct((E, H), _f32),
    )(edge_attr, we, be.reshape(1, H), wl, bl.reshape(1, H))


def _node_body(h_ref, p0_ref, p1_ref, w1_ref, b1_ref, g_ref, bt_ref, m_ref,
               v_ref, w2_ref, b2_ref, o_ref):
    h = h_ref[...]
    z = h + p0_ref[...] + p1_ref[...]
    z = jnp.dot(z, w1_ref[...], preferred_element_type=_f32) + b1_ref[...]
    s = g_ref[...] * lax.rsqrt(v_ref[...] + 1e-5)
    z = (z - m_ref[...]) * s + bt_ref[...]
    z = jnp.maximum(z, 0.0)
    z = jnp.dot(z, w2_ref[...], preferred_element_type=_f32) + b2_ref[...]
    o_ref[...] = jnp.maximum(z, 0.0) + h


def _node_update(h, p0, p1, w1, b1, gamma, beta, mean, var, w2, b2):
    bn = 2000
    row = lambda a: a.reshape(1, H)
    return pl.pallas_call(
        _node_body,
        grid=(N // bn,),
        in_specs=[
            pl.BlockSpec((bn, H), lambda i: (i, 0)),
            pl.BlockSpec((bn, H), lambda i: (i, 0)),
            pl.BlockSpec((bn, H), lambda i: (i, 0)),
            pl.BlockSpec((H, H), lambda i: (0, 0)),
            pl.BlockSpec((1, H), lambda i: (0, 0)),
            pl.BlockSpec((1, H), lambda i: (0, 0)),
            pl.BlockSpec((1, H), lambda i: (0, 0)),
            pl.BlockSpec((1, H), lambda i: (0, 0)),
            pl.BlockSpec((1, H), lambda i: (0, 0)),
            pl.BlockSpec((H, H), lambda i: (0, 0)),
            pl.BlockSpec((1, H), lambda i: (0, 0)),
        ],
        out_specs=pl.BlockSpec((bn, H), lambda i: (i, 0)),
        out_shape=jax.ShapeDtypeStruct((N, H), _f32),
    )(h, p0, p1, w1, row(b1), row(gamma), row(beta), row(mean), row(var),
      w2, row(b2))


def _pool_body(h_ref, b_ref, wp_ref, bp_ref, o_ref, acc_ref, cnt_ref):
    i = pl.program_id(0)

    @pl.when(i == 0)
    def _():
        acc_ref[...] = jnp.zeros_like(acc_ref)
        cnt_ref[...] = jnp.zeros_like(cnt_ref)

    gids = lax.broadcasted_iota(jnp.int32, (1, G), 1)
    m = (b_ref[...] == gids).astype(_f32)
    dn = (((0,), (0,)), ((), ()))
    acc_ref[...] += lax.dot_general(m, h_ref[...], dn, preferred_element_type=_f32)
    cnt_ref[...] += lax.dot_general(m, jnp.ones_like(h_ref[...]), dn,
                                    preferred_element_type=_f32)

    @pl.when(i == pl.num_programs(0) - 1)
    def _():
        pooled = acc_ref[...] / jnp.maximum(cnt_ref[...], 1.0)
        o_ref[...] = (
            jnp.dot(pooled, wp_ref[...], preferred_element_type=_f32) + bp_ref[...]
        )


def _pool_project(h, batch, wp, bp):
    bn = 2000
    return pl.pallas_call(
        _pool_body,
        grid=(N // bn,),
        in_specs=[
            pl.BlockSpec((bn, H), lambda i: (i, 0)),
            pl.BlockSpec((bn, 1), lambda i: (i, 0)),
            pl.BlockSpec((H, P), lambda i: (0, 0)),
            pl.BlockSpec((1, P), lambda i: (0, 0)),
        ],
        out_specs=pl.BlockSpec((G, P), lambda i: (0, 0)),
        out_shape=jax.ShapeDtypeStruct((G, P), _f32),
        scratch_shapes=[
            pltpu.VMEM((G, H), _f32),
            pltpu.VMEM((G, H), _f32),
        ],
    )(h, batch.reshape(N, 1), wp, bp.reshape(1, P))


_sc_mesh = plsc.VectorSubcoreMesh(core_axis_name="c", subcore_axis_name="s")


@functools.partial(
    pl.kernel,
    out_type=jax.ShapeDtypeStruct((NC * NP, H), _f32),
    mesh=_sc_mesh,
    scratch_types=[
        [pltpu.VMEM((C,), jnp.int32)] * 2,
        [pltpu.VMEM((C,), jnp.int32)] * 4,
        [pltpu.VMEM((C, H), _f32)] * 2,
        [pltpu.VMEM((C, H), _f32)] * 2,
        pltpu.VMEM_SHARED((NP, H), _f32),
        [pltpu.SemaphoreType.DMA] * 2,
        [pltpu.SemaphoreType.DMA] * 2,
        [pltpu.SemaphoreType.DMA] * 2,
    ],
)
def _sc_aggr(h_hbm, ea_hbm, src_hbm, dst_hbm, out_hbm,
             src_v, dst_v, ea_v, gat_v, aggr, sem_idx, sem_gat, sem_sct):
    cid = lax.axis_index("c")
    sid = lax.axis_index("s")
    wid = cid * NS + sid
    base0 = wid * EPW

    def _issue_idx(g, b, ring):
        base = base0 + g * C
        pltpu.async_copy(src_hbm.at[pl.ds(base, C)], src_v[b], sem_idx[b])
        pltpu.async_copy(dst_hbm.at[pl.ds(base, C)], dst_v[ring], sem_idx[b])
        pltpu.async_copy(ea_hbm.at[pl.ds(base, C)], ea_v[b], sem_idx[b])

    def _wait_idx(g, b):
        base = base0 + g * C
        pltpu.make_async_copy(src_hbm.at[pl.ds(base, C)], src_v[b],
                              sem_idx[b]).wait()
        pltpu.make_async_copy(dst_hbm.at[pl.ds(base, C)], dst_v[0],
                              sem_idx[b]).wait()
        pltpu.make_async_copy(ea_hbm.at[pl.ds(base, C)], ea_v[b],
                              sem_idx[b]).wait()

    def _start_gather(b):
        pltpu.async_copy(h_hbm.at[src_v[b]], gat_v[b], sem_gat[b])

    def _wait_gather(b):
        pltpu.make_async_copy(h_hbm.at[src_v[b]], gat_v[b], sem_gat[b]).wait()

    def _relu(b):
        @plsc.parallel_loop(0, C, 1, unroll=2)
        def _rows(r):
            for c in range(H // 16):
                sl = pl.ds(c * 16, 16)
                gat_v[b][r, sl] = jnp.maximum(
                    gat_v[b][r, sl] + ea_v[b][r, sl], 0.0)

    def _issue_sct(b, ring):
        pltpu.async_copy(gat_v[b], aggr.at[dst_v[ring]], sem_sct[b], add=True)

    def _wait_sct(b, ring):
        pltpu.make_async_copy(gat_v[b], aggr.at[dst_v[ring]],
                              sem_sct[b]).wait()

    def _phase(g, b, ring, first=False, has1=True, has2=True):
        nb = 1 - b
        if has1:
            _wait_idx(g + 1, nb)
        if not first:
            _wait_sct(nb, (ring + 3) % 4)
        if has1:
            _start_gather(nb)
        _wait_gather(b)
        _relu(b)
        _issue_sct(b, ring)
        if has2:
            _issue_idx(g + 2, b, (ring + 2) % 4)

    _issue_idx(0, 0, 0)
    _issue_idx(1, 1, 1)

    zero16 = jnp.zeros((16,), _f32)

    def _zero_row(r, carry):
        for c in range(H // 16):
            gat_v[0][r, pl.ds(c * 16, 16)] = zero16
        return carry

    lax.fori_loop(0, C, _zero_row, 0)
    for j in range(RPT // C):
        pltpu.sync_copy(gat_v[0], aggr.at[pl.ds(sid * RPT + j * C, C)])
    plsc.subcore_barrier()

    _wait_idx(0, 0)
    _start_gather(0)

    _phase(0, 0, 0, first=True)
    _phase(1, 1, 1)
    _phase(2, 0, 2)
    _phase(3, 1, 3)

    def _quad(m, carry):
        g0 = m * 4
        _phase(g0, 0, 0)
        _phase(g0 + 1, 1, 1)
        _phase(g0 + 2, 0, 2)
        _phase(g0 + 3, 1, 3)
        return carry

    lax.fori_loop(1, (NCHUNK - 5) // 4, _quad, 0)

    _phase(NCHUNK - 5, 0, 0)
    _phase(NCHUNK - 4, 1, 1)
    _phase(NCHUNK - 3, 0, 2)
    _phase(NCHUNK - 2, 1, 3, has2=False)
    _phase(NCHUNK - 1, 0, 0, has1=False, has2=False)
    _wait_sct(0, 0)

    plsc.subcore_barrier()
    for j in range(RPT // C):
        r0 = sid * RPT + j * C
        pltpu.sync_copy(aggr.at[pl.ds(r0, C)],
                        out_hbm.at[pl.ds(cid * NP + r0, C)])


def kernel(x, edge_index, edge_attr, batch, params):
    src = edge_index[0]
    dst = edge_index[1]
    h = _embed(x, params['W_node'], params['b_node'])
    ea_ls = [_edge_linear(edge_attr, params['W_edge'], params['b_edge'],
                          params['W_lin'][i], params['b_lin'][i])
             for i in range(L)]
    for i in range(L):
        parts = _sc_aggr(h, ea_ls[i], src, dst)
        h = _node_update(h, parts[:N], parts[NP:NP + N],
                         params['W1'][i], params['b1'][i],
                         params['bn_gamma'][i], params['bn_beta'][i],
                         params['bn_mean'][i], params['bn_var'][i],
                         params['W2'][i], params['b2'][i])
    return _pool_project(h, batch, params['W_proj'], params['b_proj'])


# speedup vs baseline: 1.0094x; 1.0094x over previous
import functools

import jax
import jax.numpy as jnp
from jax import lax
from jax.experimental import pallas as pl
from jax.experimental.pallas import tpu as pltpu
from jax.experimental.pallas import tpu_sc as plsc

N = 10000
E = 320000
NODE_F = 11
EDGE_F = 14
H = 128
P = 128
L = 3
G = 16

NC = 2
NS = 16
NW = NC * NS
EPW = E // NW
C = 80
NCHUNK = EPW // C
NP = 10240
RPT = NP // NS

_f32 = jnp.float32


def _embed_body(x_ref, w_ref, b_ref, o_ref):
    o_ref[...] = (
        jnp.dot(x_ref[...], w_ref[...], preferred_element_type=_f32) + b_ref[...]
    )


def _embed(x, w, b):
    bn = 2000
    return pl.pallas_call(
        _embed_body,
        grid=(N // bn,),
        in_specs=[
            pl.BlockSpec((bn, NODE_F), lambda i: (i, 0)),
            pl.BlockSpec((NODE_F, H), lambda i: (0, 0)),
            pl.BlockSpec((1, H), lambda i: (0, 0)),
        ],
        out_specs=pl.BlockSpec((bn, H), lambda i: (i, 0)),
        out_shape=jax.ShapeDtypeStruct((N, H), _f32),
    )(x, w, b.reshape(1, H))


def _ea_body(a_ref, we_ref, be_ref, wl_ref, bl_ref, o_ref):
    t = jnp.dot(a_ref[...], we_ref[...], preferred_element_type=_f32) + be_ref[...]
    o_ref[...] = jnp.dot(t, wl_ref[...], preferred_element_type=_f32) + bl_ref[...]


def _edge_linear(edge_attr, we, be, wl, bl):
    bn = 4000
    return pl.pallas_call(
        _ea_body,
        grid=(E // bn,),
        in_specs=[
            pl.BlockSpec((bn, EDGE_F), lambda i: (i, 0)),
            pl.BlockSpec((EDGE_F, H), lambda i: (0, 0)),
            pl.BlockSpec((1, H), lambda i: (0, 0)),
            pl.BlockSpec((H, H), lambda i: (0, 0)),
            pl.BlockSpec((1, H), lambda i: (0, 0)),
        ],
        out_specs=pl.BlockSpec((bn, H), lambda i: (i, 0)),
        out_shape=jax.ShapeDtypeStruct((E, H), _f32),
    )(edge_attr, we, be.reshape(1, H), wl, bl.reshape(1, H))


def _node_body(h_ref, p0_ref, p1_ref, w1_ref, b1_ref, g_ref, bt_ref, m_ref,
               v_ref, w2_ref, b2_ref, o_ref):
    h = h_ref[...]
    z = h + p0_ref[...] + p1_ref[...]
    z = jnp.dot(z, w1_ref[...], preferred_element_type=_f32) + b1_ref[...]
    s = g_ref[...] * lax.rsqrt(v_ref[...] + 1e-5)
    z = (z - m_ref[...]) * s + bt_ref[...]
    z = jnp.maximum(z, 0.0)
    z = jnp.dot(z, w2_ref[...], preferred_element_type=_f32) + b2_ref[...]
    o_ref[...] = jnp.maximum(z, 0.0) + h


def _node_update(h, p0, p1, w1, b1, gamma, beta, mean, var, w2, b2):
    bn = 2000
    row = lambda a: a.reshape(1, H)
    return pl.pallas_call(
        _node_body,
        grid=(N // bn,),
        in_specs=[
            pl.BlockSpec((bn, H), lambda i: (i, 0)),
            pl.BlockSpec((bn, H), lambda i: (i, 0)),
            pl.BlockSpec((bn, H), lambda i: (i, 0)),
            pl.BlockSpec((H, H), lambda i: (0, 0)),
            pl.BlockSpec((1, H), lambda i: (0, 0)),
            pl.BlockSpec((1, H), lambda i: (0, 0)),
            pl.BlockSpec((1, H), lambda i: (0, 0)),
            pl.BlockSpec((1, H), lambda i: (0, 0)),
            pl.BlockSpec((1, H), lambda i: (0, 0)),
            pl.BlockSpec((H, H), lambda i: (0, 0)),
            pl.BlockSpec((1, H), lambda i: (0, 0)),
        ],
        out_specs=pl.BlockSpec((bn, H), lambda i: (i, 0)),
        out_shape=jax.ShapeDtypeStruct((N, H), _f32),
    )(h, p0, p1, w1, row(b1), row(gamma), row(beta), row(mean), row(var),
      w2, row(b2))


def _pool_body(h_ref, b_ref, wp_ref, bp_ref, o_ref, acc_ref, cnt_ref):
    i = pl.program_id(0)

    @pl.when(i == 0)
    def _():
        acc_ref[...] = jnp.zeros_like(acc_ref)
        cnt_ref[...] = jnp.zeros_like(cnt_ref)

    gids = lax.broadcasted_iota(jnp.int32, (1, G), 1)
    m = (b_ref[...] == gids).astype(_f32)
    dn = (((0,), (0,)), ((), ()))
    acc_ref[...] += lax.dot_general(m, h_ref[...], dn, preferred_element_type=_f32)
    cnt_ref[...] += lax.dot_general(m, jnp.ones_like(h_ref[...]), dn,
                                    preferred_element_type=_f32)

    @pl.when(i == pl.num_programs(0) - 1)
    def _():
        pooled = acc_ref[...] / jnp.maximum(cnt_ref[...], 1.0)
        o_ref[...] = (
            jnp.dot(pooled, wp_ref[...], preferred_element_type=_f32) + bp_ref[...]
        )


def _pool_project(h, batch, wp, bp):
    bn = 2000
    return pl.pallas_call(
        _pool_body,
        grid=(N // bn,),
        in_specs=[
            pl.BlockSpec((bn, H), lambda i: (i, 0)),
            pl.BlockSpec((bn, 1), lambda i: (i, 0)),
            pl.BlockSpec((H, P), lambda i: (0, 0)),
            pl.BlockSpec((1, P), lambda i: (0, 0)),
        ],
        out_specs=pl.BlockSpec((G, P), lambda i: (0, 0)),
        out_shape=jax.ShapeDtypeStruct((G, P), _f32),
        scratch_shapes=[
            pltpu.VMEM((G, H), _f32),
            pltpu.VMEM((G, H), _f32),
        ],
    )(h, batch.reshape(N, 1), wp, bp.reshape(1, P))


_sc_mesh = plsc.VectorSubcoreMesh(core_axis_name="c", subcore_axis_name="s")


@functools.partial(
    pl.kernel,
    out_type=jax.ShapeDtypeStruct((NC * NP, H), _f32),
    mesh=_sc_mesh,
    scratch_types=[
        [pltpu.VMEM((C,), jnp.int32)] * 2,
        [pltpu.VMEM((C,), jnp.int32)] * 4,
        [pltpu.VMEM((C, H), _f32)] * 2,
        [pltpu.VMEM((C, H), _f32)] * 2,
        pltpu.VMEM_SHARED((NP, H), _f32),
        [pltpu.SemaphoreType.DMA] * 2,
        [pltpu.SemaphoreType.DMA] * 2,
        [pltpu.SemaphoreType.DMA] * 2,
    ],
)
def _sc_aggr(h_hbm, ea_hbm, src_hbm, dst_hbm, out_hbm,
             src_v, dst_v, ea_v, gat_v, aggr, sem_idx, sem_gat, sem_sct):
    cid = lax.axis_index("c")
    sid = lax.axis_index("s")
    wid = cid * NS + sid
    base0 = wid * EPW

    def _issue_idx(g, b, ring):
        base = base0 + g * C
        pltpu.async_copy(src_hbm.at[pl.ds(base, C)], src_v[b], sem_idx[b])
        pltpu.async_copy(dst_hbm.at[pl.ds(base, C)], dst_v[ring], sem_idx[b])
        pltpu.async_copy(ea_hbm.at[pl.ds(base, C)], ea_v[b], sem_idx[b])

    def _wait_idx(g, b):
        base = base0 + g * C
        pltpu.make_async_copy(src_hbm.at[pl.ds(base, C)], src_v[b],
                              sem_idx[b]).wait()
        pltpu.make_async_copy(dst_hbm.at[pl.ds(base, C)], dst_v[0],
                              sem_idx[b]).wait()
        pltpu.make_async_copy(ea_hbm.at[pl.ds(base, C)], ea_v[b],
                              sem_idx[b]).wait()

    def _start_gather(b):
        pltpu.async_copy(h_hbm.at[src_v[b]], gat_v[b], sem_gat[b])

    def _wait_gather(b):
        pltpu.make_async_copy(h_hbm.at[src_v[b]], gat_v[b], sem_gat[b]).wait()

    def _relu(b):
        def _rows(r, rc):
            for u in range(2):
                for c in range(H // 16):
                    sl = pl.ds(c * 16, 16)
                    gat_v[b][r * 2 + u, sl] = jnp.maximum(
                        gat_v[b][r * 2 + u, sl] + ea_v[b][r * 2 + u, sl], 0.0)
            return rc

        lax.fori_loop(0, C // 2, _rows, 0)

    def _issue_sct(b, ring):
        pltpu.async_copy(gat_v[b], aggr.at[dst_v[ring]], sem_sct[b], add=True)

    def _wait_sct(b, ring):
        pltpu.make_async_copy(gat_v[b], aggr.at[dst_v[ring]],
                              sem_sct[b]).wait()

    def _phase(g, b, ring, first=False, has1=True, has2=True):
        nb = 1 - b
        if has1:
            _wait_idx(g + 1, nb)
        if not first:
            _wait_sct(nb, (ring + 3) % 4)
        if has1:
            _start_gather(nb)
        _wait_gather(b)
        _relu(b)
        _issue_sct(b, ring)
        if has2:
            _issue_idx(g + 2, b, (ring + 2) % 4)

    _issue_idx(0, 0, 0)
    _issue_idx(1, 1, 1)

    zero16 = jnp.zeros((16,), _f32)

    def _zero_row(r, carry):
        for c in range(H // 16):
            gat_v[0][r, pl.ds(c * 16, 16)] = zero16
        return carry

    lax.fori_loop(0, C, _zero_row, 0)
    for j in range(RPT // C):
        pltpu.sync_copy(gat_v[0], aggr.at[pl.ds(sid * RPT + j * C, C)])
    plsc.subcore_barrier()

    _wait_idx(0, 0)
    _start_gather(0)

    _phase(0, 0, 0, first=True)
    _phase(1, 1, 1)
    _phase(2, 0, 2)
    _phase(3, 1, 3)

    def _quad(m, carry):
        g0 = m * 4
        _phase(g0, 0, 0)
        _phase(g0 + 1, 1, 1)
        _phase(g0 + 2, 0, 2)
        _phase(g0 + 3, 1, 3)
        return carry

    lax.fori_loop(1, (NCHUNK - 5) // 4, _quad, 0)

    _phase(NCHUNK - 5, 0, 0)
    _phase(NCHUNK - 4, 1, 1)
    _phase(NCHUNK - 3, 0, 2)
    _phase(NCHUNK - 2, 1, 3, has2=False)
    _phase(NCHUNK - 1, 0, 0, has1=False, has2=False)
    _wait_sct(0, 0)

    plsc.subcore_barrier()
    for j in range(RPT // C):
        r0 = sid * RPT + j * C
        pltpu.sync_copy(aggr.at[pl.ds(r0, C)],
                        out_hbm.at[pl.ds(cid * NP + r0, C)])


def kernel(x, edge_index, edge_attr, batch, params):
    src = edge_index[0]
    dst = edge_index[1]
    h = _embed(x, params['W_node'], params['b_node'])
    ea_ls = [_edge_linear(edge_attr, params['W_edge'], params['b_edge'],
                          params['W_lin'][i], params['b_lin'][i])
             for i in range(L)]
    for i in range(L):
        parts = _sc_aggr(h, ea_ls[i], src, dst)
        h = _node_update(h, parts[:N], parts[NP:NP + N],
                         params['W1'][i], params['b1'][i],
                         params['bn_gamma'][i], params['bn_beta'][i],
                         params['bn_mean'][i], params['bn_var'][i],
                         params['W2'][i], params['b2'][i])
    return _pool_project(h, batch, params['W_proj'], params['b_proj'])
